# baseline (speedup 1.0000x reference)
import functools

import jax
import jax.numpy as jnp
from jax import lax
from jax.experimental import pallas as pl
from jax.experimental.pallas import tpu as pltpu
from jax.experimental.pallas import tpu_sc as plsc

_N = 10000
_NP = 10240
_E = 320000
_W = 112
_NWIN = 3072
_EP = _NWIN * _W
_D = 256
_H = 128
_STRIPE = _NP // 16
_WPT = _NWIN // 16
_CH = 6
_NCH = _WPT // _CH

_mesh = plsc.VectorSubcoreMesh(
    core_axis_name="c", subcore_axis_name="s", num_cores=2, num_subcores=16
)
_sc_params = pltpu.CompilerParams(use_tc_tiling_on_sc=False)


_WPT_D = _NWIN // 32


@functools.partial(
    pl.kernel,
    out_type=(
        jax.ShapeDtypeStruct((2 * _NP,), jnp.float32),
        jax.ShapeDtypeStruct((2 * _NP,), jnp.float32),
    ),
    mesh=_mesh,
    compiler_params=_sc_params,
    scratch_types=[
        pltpu.VMEM_SHARED((_NP,), jnp.float32),
        pltpu.VMEM_SHARED((_NP,), jnp.float32),
        pltpu.VMEM((_WPT_D, _W), jnp.int32),
        pltpu.VMEM((_WPT_D, _W), jnp.int32),
        pltpu.VMEM((_W,), jnp.float32),
        pltpu.VMEM((_STRIPE,), jnp.float32),
    ],
)
def _sc_degrees(src_hbm, dst_hbm, ones_hbm, zer_hbm, din_hbm, dout_hbm,
                din_s, dout_s, src_v, dst_v, ones_v, zer_v):
    c = lax.axis_index("c")
    s = lax.axis_index("s")
    base_w = c * (_NWIN // 2) + s * _WPT_D
    pltpu.sync_copy(ones_hbm, ones_v)
    pltpu.sync_copy(zer_hbm, zer_v)
    pltpu.sync_copy(src_hbm.at[pl.ds(base_w, _WPT_D)], src_v)
    pltpu.sync_copy(dst_hbm.at[pl.ds(base_w, _WPT_D)], dst_v)
    pltpu.sync_copy(zer_v, din_s.at[pl.ds(s * _STRIPE, _STRIPE)])
    pltpu.sync_copy(zer_v, dout_s.at[pl.ds(s * _STRIPE, _STRIPE)])
    plsc.subcore_barrier()

    def body(w, carry):
        pltpu.sync_copy(ones_v, din_s.at[dst_v.at[w]], add=True)
        pltpu.sync_copy(ones_v, dout_s.at[src_v.at[w]], add=True)
        return carry

    lax.fori_loop(0, _WPT_D, body, 0)
    plsc.subcore_barrier()
    pltpu.sync_copy(din_s.at[pl.ds(s * _STRIPE, _STRIPE)],
                    din_hbm.at[pl.ds(c * _NP + s * _STRIPE, _STRIPE)])
    pltpu.sync_copy(dout_s.at[pl.ds(s * _STRIPE, _STRIPE)],
                    dout_hbm.at[pl.ds(c * _NP + s * _STRIPE, _STRIPE)])


@functools.partial(
    pl.kernel,
    out_type=(
        jax.ShapeDtypeStruct((_NP, _H), jnp.float32),
        jax.ShapeDtypeStruct((_NP, _H), jnp.float32),
    ),
    mesh=_mesh,
    compiler_params=_sc_params,
    scratch_types=[
        pltpu.VMEM_SHARED((_NP, _H), jnp.float32),
        pltpu.VMEM((_W, _H), jnp.float32),
        pltpu.VMEM((_W, _H), jnp.float32),
        pltpu.VMEM((_W, _H), jnp.float32),
        pltpu.VMEM((_CH, _W), jnp.int32),
        pltpu.VMEM((_CH, _W), jnp.int32),
        pltpu.VMEM((_CH, _W), jnp.int32),
        pltpu.VMEM((_CH, _W), jnp.int32),
        pltpu.SemaphoreType.DMA,
        pltpu.SemaphoreType.DMA,
        pltpu.SemaphoreType.DMA,
        pltpu.SemaphoreType.DMA,
        pltpu.SemaphoreType.DMA,
    ],
)
def _sc_aggregate(hl_hbm, hr_hbm, src_hbm, dst_hbm, zer_hbm, outl_hbm, outr_hbm,
                  acc_s, rows0, rows1, rows2, srcc0, srcc1, dstc0, dstc1,
                  sem0, sem1, sem2, semi0, semi1):
    c = lax.axis_index("c")
    s = lax.axis_index("s")
    base_w = s * _WPT

    pltpu.sync_copy(zer_hbm, rows0)

    def zbody(j, carry):
        pltpu.sync_copy(rows0, acc_s.at[pl.ds(s * _STRIPE + j * _W, _W)])
        return carry

    lax.fori_loop(0, _STRIPE // _W, zbody, 0)
    pltpu.sync_copy(rows0.at[pl.ds(0, _STRIPE - (_STRIPE // _W) * _W)],
                    acc_s.at[pl.ds(s * _STRIPE + (_STRIPE // _W) * _W,
                                   _STRIPE - (_STRIPE // _W) * _W)])
    plsc.subcore_barrier()

    idx_slots = ((srcc0, dstc0, semi0), (srcc1, dstc1, semi1))
    row_bufs = ((rows0, sem0), (rows1, sem1), (rows2, sem2))

    def idx_start(q, slot):
        sv, dv, sem = idx_slots[slot]
        pltpu.async_copy(src_hbm.at[pl.ds(base_w + q * _CH, _CH)], sv, sem)
        pltpu.async_copy(dst_hbm.at[pl.ds(base_w + q * _CH, _CH)], dv, sem)

    def idx_wait(q, slot):
        sv, dv, sem = idx_slots[slot]
        pltpu.make_async_copy(src_hbm.at[pl.ds(base_w + q * _CH, _CH)],
                              sv, sem).wait()
        pltpu.make_async_copy(dst_hbm.at[pl.ds(base_w + q * _CH, _CH)],
                              dv, sem).wait()

    def gather_start(iv, p):
        rv, sem = row_bufs[p]

        @pl.when(c == 0)
        def _():
            pltpu.async_copy(hl_hbm.at[iv], rv, sem)

        @pl.when(c == 1)
        def _():
            pltpu.async_copy(hr_hbm.at[iv], rv, sem)

    def gather_wait(iv, p):
        rv, sem = row_bufs[p]

        @pl.when(c == 0)
        def _():
            pltpu.make_async_copy(hl_hbm.at[iv], rv, sem).wait()

        @pl.when(c == 1)
        def _():
            pltpu.make_async_copy(hr_hbm.at[iv], rv, sem).wait()

    idx_start(0, 0)
    idx_wait(0, 0)
    gather_start(srcc0.at[0], 0)
    gather_start(srcc0.at[1], 1)

    def chunk_body(q, slot):
        sv, dv, _ = idx_slots[slot]
        svn = idx_slots[1 - slot][0]

        for b in range(_CH):
            p = b % 3
            w = q * _CH + b
            gather_wait(sv.at[b], p)

            if b == 0:
                @pl.when(q + 1 < _NCH)
                def _():
                    idx_start(q + 1, 1 - slot)

            pm1 = (p + 2) % 3
            if b < _CH - 2:
                @pl.when(w + 2 < _WPT)
                def _():
                    gather_start(sv.at[b + 2], pm1)
            else:
                if b == _CH - 2:
                    @pl.when(q + 1 < _NCH)
                    def _():
                        idx_wait(q + 1, 1 - slot)

                @pl.when(w + 2 < _WPT)
                def _():
                    gather_start(svn.at[b + 2 - _CH], pm1)

            pltpu.sync_copy(row_bufs[p][0], acc_s.at[dv.at[b]], add=True)

    def qbody(qq, carry):
        for slot in (0, 1):
            chunk_body(2 * qq + slot, slot)
        return carry

    lax.fori_loop(0, _NCH // 2, qbody, 0)
    plsc.subcore_barrier()

    @pl.when(c == 0)
    def _():
        pltpu.sync_copy(acc_s.at[pl.ds(s * _STRIPE, _STRIPE)],
                        outl_hbm.at[pl.ds(s * _STRIPE, _STRIPE)])

    @pl.when(c == 1)
    def _():
        pltpu.sync_copy(acc_s.at[pl.ds(s * _STRIPE, _STRIPE)],
                        outr_hbm.at[pl.ds(s * _STRIPE, _STRIPE)])


_R = 1024


def _mm0_body(feat_ref, w0_ref, out_ref):
    out_ref[...] = jnp.dot(feat_ref[...], w0_ref[...],
                           preferred_element_type=jnp.float32)


def _tc_matmul0(feat, w0):
    return pl.pallas_call(
        _mm0_body,
        grid=(_NP // _R,),
        in_specs=[
            pl.BlockSpec((_R, _D), lambda i: (i, 0)),
            pl.BlockSpec((_D, _D), lambda i: (0, 0)),
        ],
        out_specs=pl.BlockSpec((_R, _D), lambda i: (i, 0)),
        out_shape=jax.ShapeDtypeStruct((_NP, _D), jnp.float32),
    )(feat, w0)


def _prescale_body(h0_ref, din_ref, dout_ref, sin_ref, sout_ref,
                   hl_ref, hr_ref):
    din = din_ref[0] + din_ref[1]
    dout = dout_ref[0] + dout_ref[1]
    sin = jnp.broadcast_to(lax.rsqrt(jnp.maximum(din, 1.0)), (_R, _H))
    sout = jnp.broadcast_to(lax.rsqrt(jnp.maximum(dout, 1.0)), (_R, _H))
    sin_ref[...] = sin
    sout_ref[...] = sout
    h0 = h0_ref[...]
    hl_ref[...] = h0[:, :_H] * sout
    hr_ref[...] = h0[:, _H:] * sout


def _tc_prescale(h0, din, dout):
    return pl.pallas_call(
        _prescale_body,
        grid=(_NP // _R,),
        in_specs=[
            pl.BlockSpec((_R, _D), lambda i: (i, 0)),
            pl.BlockSpec((2, _R, 1), lambda i: (0, i, 0)),
            pl.BlockSpec((2, _R, 1), lambda i: (0, i, 0)),
        ],
        out_specs=[pl.BlockSpec((_R, _H), lambda i: (i, 0))] * 4,
        out_shape=[jax.ShapeDtypeStruct((_NP, _H), jnp.float32)] * 4,
    )(h0, din, dout)


def _mid_body(aggl_ref, aggr_ref, sin_ref, sout_ref, b0_ref,
              hxl_ref, hxr_ref, hxpl_ref, hxpr_ref):
    sin = sin_ref[...]
    sout = sout_ref[...]
    b0 = b0_ref[...]
    hxl = jnp.maximum(aggl_ref[...] * sin + b0[:, :_H], 0.0)
    hxr = jnp.maximum(aggr_ref[...] * sin + b0[:, _H:], 0.0)
    hxl_ref[...] = hxl
    hxr_ref[...] = hxr
    hxpl_ref[...] = hxl * sout
    hxpr_ref[...] = hxr * sout


def _tc_mid(aggl, aggr, sin, sout, b0):
    return pl.pallas_call(
        _mid_body,
        grid=(_NP // _R,),
        in_specs=[
            pl.BlockSpec((_R, _H), lambda i: (i, 0)),
            pl.BlockSpec((_R, _H), lambda i: (i, 0)),
            pl.BlockSpec((_R, _H), lambda i: (i, 0)),
            pl.BlockSpec((_R, _H), lambda i: (i, 0)),
            pl.BlockSpec((1, _D), lambda i: (0, 0)),
        ],
        out_specs=[pl.BlockSpec((_R, _H), lambda i: (i, 0))] * 4,
        out_shape=[jax.ShapeDtypeStruct((_NP, _H), jnp.float32)] * 4,
    )(aggl, aggr, sin, sout, b0)


def _final_body(hxl_ref, hxr_ref, aggl_ref, aggr_ref, sin_ref, w1_ref, b1_ref,
                out_ref):
    w1 = w1_ref[...]
    sin = sin_ref[...]
    acc = jnp.dot(hxl_ref[...], w1[:_H], preferred_element_type=jnp.float32)
    acc += jnp.dot(hxr_ref[...], w1[_H:_D], preferred_element_type=jnp.float32)
    acc += jnp.dot(aggl_ref[...] * sin, w1[_D:_D + _H],
                   preferred_element_type=jnp.float32)
    acc += jnp.dot(aggr_ref[...] * sin, w1[_D + _H:],
                   preferred_element_type=jnp.float32)
    out_ref[...] = jnp.maximum(acc + b1_ref[...], 0.0)


def _tc_final(hxl, hxr, aggl, aggr, sin, w1, b1):
    return pl.pallas_call(
        _final_body,
        grid=(_NP // _R,),
        in_specs=[
            pl.BlockSpec((_R, _H), lambda i: (i, 0)),
            pl.BlockSpec((_R, _H), lambda i: (i, 0)),
            pl.BlockSpec((_R, _H), lambda i: (i, 0)),
            pl.BlockSpec((_R, _H), lambda i: (i, 0)),
            pl.BlockSpec((_R, _H), lambda i: (i, 0)),
            pl.BlockSpec((2 * _D, _H), lambda i: (0, 0)),
            pl.BlockSpec((1, _H), lambda i: (0, 0)),
        ],
        out_specs=pl.BlockSpec((_R, _H), lambda i: (i, 0)),
        out_shape=jax.ShapeDtypeStruct((_NP, _H), jnp.float32),
    )(hxl, hxr, aggl, aggr, sin, w1, b1)


def kernel(feat, edge_index, W0, b0, W1, b1):
    pad = jnp.full((2, _EP - _E), _N, jnp.int32)
    edge_p = jnp.concatenate([edge_index, pad], axis=1)
    src = edge_p[0].reshape(_NWIN, _W)
    dst = edge_p[1].reshape(_NWIN, _W)
    feat_p = jnp.pad(feat, ((0, _NP - _N), (0, 0)))
    ones_w = jnp.ones((_W,), jnp.float32)
    zer_1d = jnp.zeros((_STRIPE,), jnp.float32)
    zer_2d = jnp.zeros((_W, _H), jnp.float32)
    b0_2d = b0.reshape(1, _D)
    b1_2d = b1.reshape(1, _H)

    h0 = _tc_matmul0(feat_p, W0)
    din_p, dout_p = _sc_degrees(src, dst, ones_w, zer_1d)
    sin, sout, hl, hr = _tc_prescale(
        h0, din_p.reshape(2, _NP, 1), dout_p.reshape(2, _NP, 1))
    aggl0, aggr0 = _sc_aggregate(hl, hr, src, dst, zer_2d)
    hxl, hxr, hxpl, hxpr = _tc_mid(aggl0, aggr0, sin, sout, b0_2d)
    aggl1, aggr1 = _sc_aggregate(hxpl, hxpr, src, dst, zer_2d)
    out = _tc_final(hxl, hxr, aggl1, aggr1, sin, W1, b1_2d)
    return out[:_N]


# speedup vs baseline: 2.1461x; 2.1461x over previous
import functools

import jax
import jax.numpy as jnp
from jax import lax
from jax.experimental import pallas as pl
from jax.experimental.pallas import tpu as pltpu
from jax.experimental.pallas import tpu_sc as plsc

_N = 10000
_NP = 10112
_E = 320000
_W = 128
_NWIN = 2592
_EP = _NWIN * _W
_D = 256
_H = 128
_STRIPE = _NP // 16
_WPT = _NWIN // 16

_mesh = plsc.VectorSubcoreMesh(
    core_axis_name="c", subcore_axis_name="s", num_cores=2, num_subcores=16
)
_sc_params = pltpu.CompilerParams(use_tc_tiling_on_sc=False)


_WPT_D = _NWIN // 32


@functools.partial(
    pl.kernel,
    out_type=(
        jax.ShapeDtypeStruct((2 * _NP,), jnp.float32),
        jax.ShapeDtypeStruct((2 * _NP,), jnp.float32),
    ),
    mesh=_mesh,
    compiler_params=_sc_params,
    scratch_types=[
        pltpu.VMEM_SHARED((_NP,), jnp.float32),
        pltpu.VMEM_SHARED((_NP,), jnp.float32),
        pltpu.VMEM((_WPT_D, _W), jnp.int32),
        pltpu.VMEM((_WPT_D, _W), jnp.int32),
        pltpu.VMEM((_W,), jnp.float32),
        pltpu.VMEM((_STRIPE,), jnp.float32),
    ],
)
def _sc_degrees(src_hbm, dst_hbm, ones_hbm, zer_hbm, din_hbm, dout_hbm,
                din_s, dout_s, src_v, dst_v, ones_v, zer_v):
    c = lax.axis_index("c")
    s = lax.axis_index("s")
    base_w = c * (_NWIN // 2) + s * _WPT_D
    pltpu.sync_copy(ones_hbm, ones_v)
    pltpu.sync_copy(zer_hbm, zer_v)
    pltpu.sync_copy(src_hbm.at[pl.ds(base_w, _WPT_D)], src_v)
    pltpu.sync_copy(dst_hbm.at[pl.ds(base_w, _WPT_D)], dst_v)
    pltpu.sync_copy(zer_v, din_s.at[pl.ds(s * _STRIPE, _STRIPE)])
    pltpu.sync_copy(zer_v, dout_s.at[pl.ds(s * _STRIPE, _STRIPE)])
    plsc.subcore_barrier()

    def body(w, carry):
        pltpu.sync_copy(ones_v, din_s.at[dst_v.at[w]], add=True)
        pltpu.sync_copy(ones_v, dout_s.at[src_v.at[w]], add=True)
        return carry

    lax.fori_loop(0, _WPT_D, body, 0)
    plsc.subcore_barrier()
    pltpu.sync_copy(din_s.at[pl.ds(s * _STRIPE, _STRIPE)],
                    din_hbm.at[pl.ds(c * _NP + s * _STRIPE, _STRIPE)])
    pltpu.sync_copy(dout_s.at[pl.ds(s * _STRIPE, _STRIPE)],
                    dout_hbm.at[pl.ds(c * _NP + s * _STRIPE, _STRIPE)])


@functools.partial(
    pl.kernel,
    out_type=(
        jax.ShapeDtypeStruct((_NP, _H), jnp.float32),
        jax.ShapeDtypeStruct((_NP, _H), jnp.float32),
    ),
    mesh=_mesh,
    compiler_params=_sc_params,
    scratch_types=[
        pltpu.VMEM_SHARED((_NP, _H), jnp.float32),
        pltpu.VMEM((_W, _H), jnp.float32),
        pltpu.VMEM((_W, _H), jnp.float32),
        pltpu.VMEM((_W, _H), jnp.float32),
        pltpu.VMEM((_W,), jnp.int32),
        pltpu.VMEM((_W,), jnp.int32),
        pltpu.VMEM((_W,), jnp.int32),
        pltpu.VMEM((_W,), jnp.int32),
        pltpu.VMEM((_W,), jnp.int32),
        pltpu.VMEM((_W,), jnp.int32),
        pltpu.SemaphoreType.DMA,
        pltpu.SemaphoreType.DMA,
        pltpu.SemaphoreType.DMA,
        pltpu.SemaphoreType.DMA,
        pltpu.SemaphoreType.DMA,
        pltpu.SemaphoreType.DMA,
    ],
)
def _sc_aggregate(hl_hbm, hr_hbm, src_hbm, dst_hbm, zer_hbm, outl_hbm, outr_hbm,
                  acc_s, rows0, rows1, rows2, src0, src1, src2,
                  dst0, dst1, dst2, sem0, sem1, sem2, isem0, isem1, isem2):
    c = lax.axis_index("c")
    s = lax.axis_index("s")
    base_w = s * _WPT

    pltpu.sync_copy(zer_hbm, rows0)
    _full = (_STRIPE // _W) * _W

    def zbody(j, carry):
        pltpu.sync_copy(rows0, acc_s.at[pl.ds(s * _STRIPE + j * _W, _W)])
        return carry

    lax.fori_loop(0, _STRIPE // _W, zbody, 0)
    pltpu.sync_copy(rows0.at[pl.ds(0, _STRIPE - _full)],
                    acc_s.at[pl.ds(s * _STRIPE + _full, _STRIPE - _full)])
    plsc.subcore_barrier()

    slots = (
        (rows0, src0, dst0, sem0, isem0),
        (rows1, src1, dst1, sem1, isem1),
        (rows2, src2, dst2, sem2, isem2),
    )

    def idx_start(w, p):
        _, sv, dv, _, isem = slots[p]
        pltpu.async_copy(src_hbm.at[base_w + w], sv, isem)
        pltpu.async_copy(dst_hbm.at[base_w + w], dv, isem)

    def idx_wait(w, p):
        _, sv, dv, _, isem = slots[p]
        pltpu.make_async_copy(src_hbm.at[base_w + w], sv, isem).wait()
        pltpu.make_async_copy(dst_hbm.at[base_w + w], dv, isem).wait()

    def gather_start(p):
        rv, sv, _, sem, _ = slots[p]

        @pl.when(c == 0)
        def _():
            pltpu.async_copy(hl_hbm.at[sv], rv, sem)

        @pl.when(c == 1)
        def _():
            pltpu.async_copy(hr_hbm.at[sv], rv, sem)

    def gather_wait(p):
        rv, sv, _, sem, _ = slots[p]

        @pl.when(c == 0)
        def _():
            pltpu.make_async_copy(hl_hbm.at[sv], rv, sem).wait()

        @pl.when(c == 1)
        def _():
            pltpu.make_async_copy(hr_hbm.at[sv], rv, sem).wait()

    idx_start(0, 0)
    idx_start(1, 1)
    idx_start(2, 2)
    idx_wait(0, 0)
    gather_start(0)
    idx_wait(1, 1)
    gather_start(1)

    def body(t, carry):
        for b in range(3):
            p = b
            pm1 = (b + 2) % 3
            w = 3 * t + b
            gather_wait(p)

            @pl.when(w + 2 < _WPT)
            def _():
                idx_wait(w + 2, pm1)
                gather_start(pm1)

            rv, _, dv, _, _ = slots[p]
            pltpu.sync_copy(rv, acc_s.at[dv], add=True)

            @pl.when(w + 3 < _WPT)
            def _():
                idx_start(w + 3, p)

        return carry

    lax.fori_loop(0, _WPT // 3, body, 0)
    plsc.subcore_barrier()

    @pl.when(c == 0)
    def _():
        pltpu.sync_copy(acc_s.at[pl.ds(s * _STRIPE, _STRIPE)],
                        outl_hbm.at[pl.ds(s * _STRIPE, _STRIPE)])

    @pl.when(c == 1)
    def _():
        pltpu.sync_copy(acc_s.at[pl.ds(s * _STRIPE, _STRIPE)],
                        outr_hbm.at[pl.ds(s * _STRIPE, _STRIPE)])


_R = 632


def _mm0_body(feat_ref, w0_ref, out_ref):
    out_ref[...] = jnp.dot(feat_ref[...], w0_ref[...],
                           preferred_element_type=jnp.float32)


def _tc_matmul0(feat, w0):
    return pl.pallas_call(
        _mm0_body,
        grid=(_NP // _R,),
        in_specs=[
            pl.BlockSpec((_R, _D), lambda i: (i, 0)),
            pl.BlockSpec((_D, _D), lambda i: (0, 0)),
        ],
        out_specs=pl.BlockSpec((_R, _D), lambda i: (i, 0)),
        out_shape=jax.ShapeDtypeStruct((_NP, _D), jnp.float32),
    )(feat, w0)


def _prescale_body(h0_ref, din_ref, dout_ref, sin_ref, sout_ref,
                   hl_ref, hr_ref):
    din = din_ref[0] + din_ref[1]
    dout = dout_ref[0] + dout_ref[1]
    sin = jnp.broadcast_to(lax.rsqrt(jnp.maximum(din, 1.0)), (_R, _H))
    sout = jnp.broadcast_to(lax.rsqrt(jnp.maximum(dout, 1.0)), (_R, _H))
    sin_ref[...] = sin
    sout_ref[...] = sout
    h0 = h0_ref[...]
    hl_ref[...] = h0[:, :_H] * sout
    hr_ref[...] = h0[:, _H:] * sout


def _tc_prescale(h0, din, dout):
    return pl.pallas_call(
        _prescale_body,
        grid=(_NP // _R,),
        in_specs=[
            pl.BlockSpec((_R, _D), lambda i: (i, 0)),
            pl.BlockSpec((2, _R, 1), lambda i: (0, i, 0)),
            pl.BlockSpec((2, _R, 1), lambda i: (0, i, 0)),
        ],
        out_specs=[pl.BlockSpec((_R, _H), lambda i: (i, 0))] * 4,
        out_shape=[jax.ShapeDtypeStruct((_NP, _H), jnp.float32)] * 4,
    )(h0, din, dout)


def _mid_body(aggl_ref, aggr_ref, sin_ref, sout_ref, b0_ref,
              hxl_ref, hxr_ref, hxpl_ref, hxpr_ref):
    sin = sin_ref[...]
    sout = sout_ref[...]
    b0 = b0_ref[...]
    hxl = jnp.maximum(aggl_ref[...] * sin + b0[:, :_H], 0.0)
    hxr = jnp.maximum(aggr_ref[...] * sin + b0[:, _H:], 0.0)
    hxl_ref[...] = hxl
    hxr_ref[...] = hxr
    hxpl_ref[...] = hxl * sout
    hxpr_ref[...] = hxr * sout


def _tc_mid(aggl, aggr, sin, sout, b0):
    return pl.pallas_call(
        _mid_body,
        grid=(_NP // _R,),
        in_specs=[
            pl.BlockSpec((_R, _H), lambda i: (i, 0)),
            pl.BlockSpec((_R, _H), lambda i: (i, 0)),
            pl.BlockSpec((_R, _H), lambda i: (i, 0)),
            pl.BlockSpec((_R, _H), lambda i: (i, 0)),
            pl.BlockSpec((1, _D), lambda i: (0, 0)),
        ],
        out_specs=[pl.BlockSpec((_R, _H), lambda i: (i, 0))] * 4,
        out_shape=[jax.ShapeDtypeStruct((_NP, _H), jnp.float32)] * 4,
    )(aggl, aggr, sin, sout, b0)


def _final_body(hxl_ref, hxr_ref, aggl_ref, aggr_ref, sin_ref, w1_ref, b1_ref,
                out_ref):
    w1 = w1_ref[...]
    sin = sin_ref[...]
    acc = jnp.dot(hxl_ref[...], w1[:_H], preferred_element_type=jnp.float32)
    acc += jnp.dot(hxr_ref[...], w1[_H:_D], preferred_element_type=jnp.float32)
    acc += jnp.dot(aggl_ref[...] * sin, w1[_D:_D + _H],
                   preferred_element_type=jnp.float32)
    acc += jnp.dot(aggr_ref[...] * sin, w1[_D + _H:],
                   preferred_element_type=jnp.float32)
    out_ref[...] = jnp.maximum(acc + b1_ref[...], 0.0)


def _tc_final(hxl, hxr, aggl, aggr, sin, w1, b1):
    return pl.pallas_call(
        _final_body,
        grid=(_NP // _R,),
        in_specs=[
            pl.BlockSpec((_R, _H), lambda i: (i, 0)),
            pl.BlockSpec((_R, _H), lambda i: (i, 0)),
            pl.BlockSpec((_R, _H), lambda i: (i, 0)),
            pl.BlockSpec((_R, _H), lambda i: (i, 0)),
            pl.BlockSpec((_R, _H), lambda i: (i, 0)),
            pl.BlockSpec((2 * _D, _H), lambda i: (0, 0)),
            pl.BlockSpec((1, _H), lambda i: (0, 0)),
        ],
        out_specs=pl.BlockSpec((_R, _H), lambda i: (i, 0)),
        out_shape=jax.ShapeDtypeStruct((_NP, _H), jnp.float32),
    )(hxl, hxr, aggl, aggr, sin, w1, b1)


def kernel(feat, edge_index, W0, b0, W1, b1):
    pad = jnp.full((2, _EP - _E), _N, jnp.int32)
    edge_p = jnp.concatenate([edge_index, pad], axis=1)
    src = edge_p[0].reshape(_NWIN, _W)
    dst = edge_p[1].reshape(_NWIN, _W)
    feat_p = jnp.pad(feat, ((0, _NP - _N), (0, 0)))
    ones_w = jnp.ones((_W,), jnp.float32)
    zer_1d = jnp.zeros((_STRIPE,), jnp.float32)
    zer_2d = jnp.zeros((_W, _H), jnp.float32)
    b0_2d = b0.reshape(1, _D)
    b1_2d = b1.reshape(1, _H)

    h0 = _tc_matmul0(feat_p, W0)
    din_p, dout_p = _sc_degrees(src, dst, ones_w, zer_1d)
    sin, sout, hl, hr = _tc_prescale(
        h0, din_p.reshape(2, _NP, 1), dout_p.reshape(2, _NP, 1))
    aggl0, aggr0 = _sc_aggregate(hl, hr, src, dst, zer_2d)
    hxl, hxr, hxpl, hxpr = _tc_mid(aggl0, aggr0, sin, sout, b0_2d)
    aggl1, aggr1 = _sc_aggregate(hxpl, hxpr, src, dst, zer_2d)
    out = _tc_final(hxl, hxr, aggl1, aggr1, sin, W1, b1_2d)
    return out[:_N]


# speedup vs baseline: 2.7035x; 1.2598x over previous
import functools

import jax
import jax.numpy as jnp
from jax import lax
from jax.experimental import pallas as pl
from jax.experimental.pallas import tpu as pltpu
from jax.experimental.pallas import tpu_sc as plsc

_N = 10000
_NP = 10240
_E = 320000
_W = 128
_NWIN = 2560
_EP = _NWIN * _W
_D = 256
_H = 128
_STRIPE = _NP // 16
_WPT = _NWIN // 16
_CH = 20
_NCH = _WPT // _CH

_mesh = plsc.VectorSubcoreMesh(
    core_axis_name="c", subcore_axis_name="s", num_cores=2, num_subcores=16
)
_sc_params = pltpu.CompilerParams(use_tc_tiling_on_sc=False)


_WPT_D = _NWIN // 32


@functools.partial(
    pl.kernel,
    out_type=(
        jax.ShapeDtypeStruct((2 * _NP,), jnp.float32),
        jax.ShapeDtypeStruct((2 * _NP,), jnp.float32),
    ),
    mesh=_mesh,
    compiler_params=_sc_params,
    scratch_types=[
        pltpu.VMEM_SHARED((_NP,), jnp.float32),
        pltpu.VMEM_SHARED((_NP,), jnp.float32),
        pltpu.VMEM((_WPT_D, _W), jnp.int32),
        pltpu.VMEM((_WPT_D, _W), jnp.int32),
        pltpu.VMEM((_W,), jnp.float32),
        pltpu.VMEM((_STRIPE,), jnp.float32),
    ],
)
def _sc_degrees(src_hbm, dst_hbm, ones_hbm, zer_hbm, din_hbm, dout_hbm,
                din_s, dout_s, src_v, dst_v, ones_v, zer_v):
    c = lax.axis_index("c")
    s = lax.axis_index("s")
    base_w = c * (_NWIN // 2) + s * _WPT_D
    pltpu.sync_copy(ones_hbm, ones_v)
    pltpu.sync_copy(zer_hbm, zer_v)
    pltpu.sync_copy(src_hbm.at[pl.ds(base_w, _WPT_D)], src_v)
    pltpu.sync_copy(dst_hbm.at[pl.ds(base_w, _WPT_D)], dst_v)
    pltpu.sync_copy(zer_v, din_s.at[pl.ds(s * _STRIPE, _STRIPE)])
    pltpu.sync_copy(zer_v, dout_s.at[pl.ds(s * _STRIPE, _STRIPE)])
    plsc.subcore_barrier()

    def body(w, carry):
        pltpu.sync_copy(ones_v, din_s.at[dst_v.at[w]], add=True)
        pltpu.sync_copy(ones_v, dout_s.at[src_v.at[w]], add=True)
        return carry

    lax.fori_loop(0, _WPT_D, body, 0)
    plsc.subcore_barrier()
    pltpu.sync_copy(din_s.at[pl.ds(s * _STRIPE, _STRIPE)],
                    din_hbm.at[pl.ds(c * _NP + s * _STRIPE, _STRIPE)])
    pltpu.sync_copy(dout_s.at[pl.ds(s * _STRIPE, _STRIPE)],
                    dout_hbm.at[pl.ds(c * _NP + s * _STRIPE, _STRIPE)])


@functools.partial(
    pl.kernel,
    out_type=(
        jax.ShapeDtypeStruct((_NP, _H), jnp.float32),
        jax.ShapeDtypeStruct((_NP, _H), jnp.float32),
    ),
    mesh=_mesh,
    compiler_params=_sc_params,
    scratch_types=[
        pltpu.VMEM_SHARED((_NP, _H), jnp.float32),
        pltpu.VMEM((_W, _H), jnp.float32),
        pltpu.VMEM((_W, _H), jnp.float32),
        pltpu.VMEM((_CH, _W), jnp.int32),
        pltpu.VMEM((_CH, _W), jnp.int32),
        pltpu.VMEM((_CH, _W), jnp.int32),
        pltpu.VMEM((_CH, _W), jnp.int32),
        pltpu.SemaphoreType.DMA,
        pltpu.SemaphoreType.DMA,
        pltpu.SemaphoreType.DMA,
        pltpu.SemaphoreType.DMA,
    ],
)
def _sc_aggregate(hl_hbm, hr_hbm, src_hbm, dst_hbm, zer_hbm, outl_hbm, outr_hbm,
                  acc_s, rows0, rows1, srcc0, srcc1, dstc0, dstc1,
                  sem0, sem1, semi0, semi1):
    c = lax.axis_index("c")
    s = lax.axis_index("s")
    base_w = s * _WPT

    pltpu.sync_copy(zer_hbm, rows0)

    def zbody(j, carry):
        pltpu.sync_copy(rows0, acc_s.at[pl.ds(s * _STRIPE + j * _W, _W)])
        return carry

    lax.fori_loop(0, _STRIPE // _W, zbody, 0)
    plsc.subcore_barrier()

    idx_slots = ((srcc0, dstc0, semi0), (srcc1, dstc1, semi1))

    def idx_start(q, slot):
        sv, dv, sem = idx_slots[slot]
        pltpu.async_copy(src_hbm.at[pl.ds(base_w + q * _CH, _CH)], sv, sem)
        pltpu.async_copy(dst_hbm.at[pl.ds(base_w + q * _CH, _CH)], dv, sem)

    def idx_wait(q, slot):
        sv, dv, sem = idx_slots[slot]
        pltpu.make_async_copy(src_hbm.at[pl.ds(base_w + q * _CH, _CH)],
                              sv, sem).wait()
        pltpu.make_async_copy(dst_hbm.at[pl.ds(base_w + q * _CH, _CH)],
                              dv, sem).wait()

    row_bufs = ((rows0, sem0), (rows1, sem1))

    def gather_start(iv, rv, sem):
        @pl.when(c == 0)
        def _():
            pltpu.async_copy(hl_hbm.at[iv], rv, sem)

        @pl.when(c == 1)
        def _():
            pltpu.async_copy(hr_hbm.at[iv], rv, sem)

    def gather_wait(iv, rv, sem):
        @pl.when(c == 0)
        def _():
            pltpu.make_async_copy(hl_hbm.at[iv], rv, sem).wait()

        @pl.when(c == 1)
        def _():
            pltpu.make_async_copy(hr_hbm.at[iv], rv, sem).wait()

    idx_start(0, 0)
    idx_start(1, 1)

    def chunk_body(sv, dv, q, slot):
        gather_start(sv.at[0], rows0, sem0)
        gather_start(sv.at[1], rows1, sem1)

        def wbody(t, carry):
            for b, (rv, sem) in enumerate(row_bufs):
                w = 2 * t + b
                gather_wait(sv.at[w], rv, sem)
                pltpu.sync_copy(rv, acc_s.at[dv.at[w]], add=True)

                @pl.when(w + 2 < _CH)
                def _():
                    gather_start(sv.at[w + 2], rv, sem)

            return carry

        lax.fori_loop(0, _CH // 2, wbody, 0)

        @pl.when(q + 2 < _NCH)
        def _():
            idx_start(q + 2, slot)

    def qbody(qq, carry):
        for slot in (0, 1):
            q = 2 * qq + slot
            sv, dv, _ = idx_slots[slot]
            idx_wait(q, slot)
            chunk_body(sv, dv, q, slot)
        return carry

    lax.fori_loop(0, _NCH // 2, qbody, 0)
    plsc.subcore_barrier()

    @pl.when(c == 0)
    def _():
        pltpu.sync_copy(acc_s.at[pl.ds(s * _STRIPE, _STRIPE)],
                        outl_hbm.at[pl.ds(s * _STRIPE, _STRIPE)])

    @pl.when(c == 1)
    def _():
        pltpu.sync_copy(acc_s.at[pl.ds(s * _STRIPE, _STRIPE)],
                        outr_hbm.at[pl.ds(s * _STRIPE, _STRIPE)])


_R = 1024


def _mm0_body(feat_ref, w0_ref, out_ref):
    out_ref[...] = jnp.dot(feat_ref[...], w0_ref[...],
                           preferred_element_type=jnp.float32)


def _tc_matmul0(feat, w0):
    return pl.pallas_call(
        _mm0_body,
        grid=(_NP // _R,),
        in_specs=[
            pl.BlockSpec((_R, _D), lambda i: (i, 0)),
            pl.BlockSpec((_D, _D), lambda i: (0, 0)),
        ],
        out_specs=pl.BlockSpec((_R, _D), lambda i: (i, 0)),
        out_shape=jax.ShapeDtypeStruct((_NP, _D), jnp.float32),
    )(feat, w0)


def _prescale_body(h0_ref, din_ref, dout_ref, sin_ref, sout_ref,
                   hl_ref, hr_ref):
    din = din_ref[0] + din_ref[1]
    dout = dout_ref[0] + dout_ref[1]
    sin = jnp.broadcast_to(lax.rsqrt(jnp.maximum(din, 1.0)), (_R, _H))
    sout = jnp.broadcast_to(lax.rsqrt(jnp.maximum(dout, 1.0)), (_R, _H))
    sin_ref[...] = sin
    sout_ref[...] = sout
    h0 = h0_ref[...]
    hl_ref[...] = h0[:, :_H] * sout
    hr_ref[...] = h0[:, _H:] * sout


def _tc_prescale(h0, din, dout):
    return pl.pallas_call(
        _prescale_body,
        grid=(_NP // _R,),
        in_specs=[
            pl.BlockSpec((_R, _D), lambda i: (i, 0)),
            pl.BlockSpec((2, _R, 1), lambda i: (0, i, 0)),
            pl.BlockSpec((2, _R, 1), lambda i: (0, i, 0)),
        ],
        out_specs=[pl.BlockSpec((_R, _H), lambda i: (i, 0))] * 4,
        out_shape=[jax.ShapeDtypeStruct((_NP, _H), jnp.float32)] * 4,
    )(h0, din, dout)


def _mid_body(aggl_ref, aggr_ref, sin_ref, sout_ref, b0_ref,
              hxl_ref, hxr_ref, hxpl_ref, hxpr_ref):
    sin = sin_ref[...]
    sout = sout_ref[...]
    b0 = b0_ref[...]
    hxl = jnp.maximum(aggl_ref[...] * sin + b0[:, :_H], 0.0)
    hxr = jnp.maximum(aggr_ref[...] * sin + b0[:, _H:], 0.0)
    hxl_ref[...] = hxl
    hxr_ref[...] = hxr
    hxpl_ref[...] = hxl * sout
    hxpr_ref[...] = hxr * sout


def _tc_mid(aggl, aggr, sin, sout, b0):
    return pl.pallas_call(
        _mid_body,
        grid=(_NP // _R,),
        in_specs=[
            pl.BlockSpec((_R, _H), lambda i: (i, 0)),
            pl.BlockSpec((_R, _H), lambda i: (i, 0)),
            pl.BlockSpec((_R, _H), lambda i: (i, 0)),
            pl.BlockSpec((_R, _H), lambda i: (i, 0)),
            pl.BlockSpec((1, _D), lambda i: (0, 0)),
        ],
        out_specs=[pl.BlockSpec((_R, _H), lambda i: (i, 0))] * 4,
        out_shape=[jax.ShapeDtypeStruct((_NP, _H), jnp.float32)] * 4,
    )(aggl, aggr, sin, sout, b0)


def _final_body(hxl_ref, hxr_ref, aggl_ref, aggr_ref, sin_ref, w1_ref, b1_ref,
                out_ref):
    w1 = w1_ref[...]
    sin = sin_ref[...]
    acc = jnp.dot(hxl_ref[...], w1[:_H], preferred_element_type=jnp.float32)
    acc += jnp.dot(hxr_ref[...], w1[_H:_D], preferred_element_type=jnp.float32)
    acc += jnp.dot(aggl_ref[...] * sin, w1[_D:_D + _H],
                   preferred_element_type=jnp.float32)
    acc += jnp.dot(aggr_ref[...] * sin, w1[_D + _H:],
                   preferred_element_type=jnp.float32)
    out_ref[...] = jnp.maximum(acc + b1_ref[...], 0.0)


def _tc_final(hxl, hxr, aggl, aggr, sin, w1, b1):
    return pl.pallas_call(
        _final_body,
        grid=(_NP // _R,),
        in_specs=[
            pl.BlockSpec((_R, _H), lambda i: (i, 0)),
            pl.BlockSpec((_R, _H), lambda i: (i, 0)),
            pl.BlockSpec((_R, _H), lambda i: (i, 0)),
            pl.BlockSpec((_R, _H), lambda i: (i, 0)),
            pl.BlockSpec((_R, _H), lambda i: (i, 0)),
            pl.BlockSpec((2 * _D, _H), lambda i: (0, 0)),
            pl.BlockSpec((1, _H), lambda i: (0, 0)),
        ],
        out_specs=pl.BlockSpec((_R, _H), lambda i: (i, 0)),
        out_shape=jax.ShapeDtypeStruct((_NP, _H), jnp.float32),
    )(hxl, hxr, aggl, aggr, sin, w1, b1)


def kernel(feat, edge_index, W0, b0, W1, b1):
    pad = jnp.full((2, _EP - _E), _N, jnp.int32)
    edge_p = jnp.concatenate([edge_index, pad], axis=1)
    src = edge_p[0].reshape(_NWIN, _W)
    dst = edge_p[1].reshape(_NWIN, _W)
    feat_p = jnp.pad(feat, ((0, _NP - _N), (0, 0)))
    ones_w = jnp.ones((_W,), jnp.float32)
    zer_1d = jnp.zeros((_STRIPE,), jnp.float32)
    zer_2d = jnp.zeros((_W, _H), jnp.float32)
    b0_2d = b0.reshape(1, _D)
    b1_2d = b1.reshape(1, _H)

    h0 = _tc_matmul0(feat_p, W0)
    din_p, dout_p = _sc_degrees(src, dst, ones_w, zer_1d)
    sin, sout, hl, hr = _tc_prescale(
        h0, din_p.reshape(2, _NP, 1), dout_p.reshape(2, _NP, 1))
    aggl0, aggr0 = _sc_aggregate(hl, hr, src, dst, zer_2d)
    hxl, hxr, hxpl, hxpr = _tc_mid(aggl0, aggr0, sin, sout, b0_2d)
    aggl1, aggr1 = _sc_aggregate(hxpl, hxpr, src, dst, zer_2d)
    out = _tc_final(hxl, hxr, aggl1, aggr1, sin, W1, b1_2d)
    return out[:_N]


# speedup vs baseline: 2.7374x; 1.0126x over previous
import functools

import jax
import jax.numpy as jnp
from jax import lax
from jax.experimental import pallas as pl
from jax.experimental.pallas import tpu as pltpu
from jax.experimental.pallas import tpu_sc as plsc

_N = 10000
_NP = 10240
_E = 320000
_W = 128
_NWIN = 2560
_EP = _NWIN * _W
_D = 256
_H = 128
_STRIPE = _NP // 16
_WPT = _NWIN // 16
_CH = 20
_NCH = _WPT // _CH

_mesh = plsc.VectorSubcoreMesh(
    core_axis_name="c", subcore_axis_name="s", num_cores=2, num_subcores=16
)
_sc_params = pltpu.CompilerParams(use_tc_tiling_on_sc=False)


_WPT_D = _NWIN // 32


@functools.partial(
    pl.kernel,
    out_type=(
        jax.ShapeDtypeStruct((2 * _NP,), jnp.float32),
        jax.ShapeDtypeStruct((2 * _NP,), jnp.float32),
    ),
    mesh=_mesh,
    compiler_params=_sc_params,
    scratch_types=[
        pltpu.VMEM_SHARED((_NP,), jnp.float32),
        pltpu.VMEM_SHARED((_NP,), jnp.float32),
        pltpu.VMEM((_WPT_D, _W), jnp.int32),
        pltpu.VMEM((_WPT_D, _W), jnp.int32),
        pltpu.VMEM((_W,), jnp.float32),
        pltpu.VMEM((_STRIPE,), jnp.float32),
    ],
)
def _sc_degrees(src_hbm, dst_hbm, ones_hbm, zer_hbm, din_hbm, dout_hbm,
                din_s, dout_s, src_v, dst_v, ones_v, zer_v):
    c = lax.axis_index("c")
    s = lax.axis_index("s")
    base_w = c * (_NWIN // 2) + s * _WPT_D
    pltpu.sync_copy(ones_hbm, ones_v)
    pltpu.sync_copy(zer_hbm, zer_v)
    pltpu.sync_copy(src_hbm.at[pl.ds(base_w, _WPT_D)], src_v)
    pltpu.sync_copy(dst_hbm.at[pl.ds(base_w, _WPT_D)], dst_v)
    pltpu.sync_copy(zer_v, din_s.at[pl.ds(s * _STRIPE, _STRIPE)])
    pltpu.sync_copy(zer_v, dout_s.at[pl.ds(s * _STRIPE, _STRIPE)])
    plsc.subcore_barrier()

    def body(w, carry):
        pltpu.sync_copy(ones_v, din_s.at[dst_v.at[w]], add=True)
        pltpu.sync_copy(ones_v, dout_s.at[src_v.at[w]], add=True)
        return carry

    lax.fori_loop(0, _WPT_D, body, 0)
    plsc.subcore_barrier()
    pltpu.sync_copy(din_s.at[pl.ds(s * _STRIPE, _STRIPE)],
                    din_hbm.at[pl.ds(c * _NP + s * _STRIPE, _STRIPE)])
    pltpu.sync_copy(dout_s.at[pl.ds(s * _STRIPE, _STRIPE)],
                    dout_hbm.at[pl.ds(c * _NP + s * _STRIPE, _STRIPE)])


@functools.partial(
    pl.kernel,
    out_type=(
        jax.ShapeDtypeStruct((_NP, _H), jnp.float32),
        jax.ShapeDtypeStruct((_NP, _H), jnp.float32),
    ),
    mesh=_mesh,
    compiler_params=_sc_params,
    scratch_types=[
        pltpu.VMEM_SHARED((_NP, _H), jnp.float32),
        pltpu.VMEM((_W, _H), jnp.float32),
        pltpu.VMEM((_W, _H), jnp.float32),
        pltpu.VMEM((_CH, _W), jnp.int32),
        pltpu.VMEM((_CH, _W), jnp.int32),
        pltpu.VMEM((_CH, _W), jnp.int32),
        pltpu.VMEM((_CH, _W), jnp.int32),
        pltpu.SemaphoreType.DMA,
        pltpu.SemaphoreType.DMA,
        pltpu.SemaphoreType.DMA,
        pltpu.SemaphoreType.DMA,
    ],
)
def _sc_aggregate(hl_hbm, hr_hbm, src_hbm, dst_hbm, zer_hbm, outl_hbm, outr_hbm,
                  acc_s, rows0, rows1, srcc0, srcc1, dstc0, dstc1,
                  sem0, sem1, semi0, semi1):
    c = lax.axis_index("c")
    s = lax.axis_index("s")
    base_w = s * _WPT

    pltpu.sync_copy(zer_hbm, rows0)

    def zbody(j, carry):
        pltpu.sync_copy(rows0, acc_s.at[pl.ds(s * _STRIPE + j * _W, _W)])
        return carry

    lax.fori_loop(0, _STRIPE // _W, zbody, 0)
    plsc.subcore_barrier()

    idx_slots = ((srcc0, dstc0, semi0), (srcc1, dstc1, semi1))

    def idx_start(q, slot):
        sv, dv, sem = idx_slots[slot]
        pltpu.async_copy(src_hbm.at[pl.ds(base_w + q * _CH, _CH)], sv, sem)
        pltpu.async_copy(dst_hbm.at[pl.ds(base_w + q * _CH, _CH)], dv, sem)

    def idx_wait(q, slot):
        sv, dv, sem = idx_slots[slot]
        pltpu.make_async_copy(src_hbm.at[pl.ds(base_w + q * _CH, _CH)],
                              sv, sem).wait()
        pltpu.make_async_copy(dst_hbm.at[pl.ds(base_w + q * _CH, _CH)],
                              dv, sem).wait()

    row_bufs = ((rows0, sem0), (rows1, sem1))

    def gather_start(iv, rv, sem):
        @pl.when(c == 0)
        def _():
            pltpu.async_copy(hl_hbm.at[iv], rv, sem)

        @pl.when(c == 1)
        def _():
            pltpu.async_copy(hr_hbm.at[iv], rv, sem)

    def gather_wait(iv, rv, sem):
        @pl.when(c == 0)
        def _():
            pltpu.make_async_copy(hl_hbm.at[iv], rv, sem).wait()

        @pl.when(c == 1)
        def _():
            pltpu.make_async_copy(hr_hbm.at[iv], rv, sem).wait()

    idx_start(0, 0)
    idx_start(1, 1)
    idx_wait(0, 0)
    gather_start(srcc0.at[0], rows0, sem0)
    gather_start(srcc0.at[1], rows1, sem1)

    def chunk_body(sv, dv, svn, q, slot):
        def wbody(t, carry):
            for b, (rv, sem) in enumerate(row_bufs):
                w = 2 * t + b
                gather_wait(sv.at[w], rv, sem)
                pltpu.sync_copy(rv, acc_s.at[dv.at[w]], add=True)

                @pl.when(w + 2 < _CH)
                def _():
                    gather_start(sv.at[w + 2], rv, sem)

                @pl.when((w + 2 >= _CH) & (q + 1 < _NCH))
                def _():
                    gather_start(svn.at[w + 2 - _CH], rv, sem)

            return carry

        lax.fori_loop(0, _CH // 2, wbody, 0)

        @pl.when(q + 2 < _NCH)
        def _():
            idx_start(q + 2, slot)

    def qbody(qq, carry):
        for slot in (0, 1):
            q = 2 * qq + slot
            sv, dv, _ = idx_slots[slot]
            svn = idx_slots[1 - slot][0]

            @pl.when(q + 1 < _NCH)
            def _():
                idx_wait(q + 1, 1 - slot)

            chunk_body(sv, dv, svn, q, slot)
        return carry

    lax.fori_loop(0, _NCH // 2, qbody, 0)
    plsc.subcore_barrier()

    @pl.when(c == 0)
    def _():
        pltpu.sync_copy(acc_s.at[pl.ds(s * _STRIPE, _STRIPE)],
                        outl_hbm.at[pl.ds(s * _STRIPE, _STRIPE)])

    @pl.when(c == 1)
    def _():
        pltpu.sync_copy(acc_s.at[pl.ds(s * _STRIPE, _STRIPE)],
                        outr_hbm.at[pl.ds(s * _STRIPE, _STRIPE)])


_R = 1024


def _mm0_body(feat_ref, w0_ref, out_ref):
    out_ref[...] = jnp.dot(feat_ref[...], w0_ref[...],
                           preferred_element_type=jnp.float32)


def _tc_matmul0(feat, w0):
    return pl.pallas_call(
        _mm0_body,
        grid=(_NP // _R,),
        in_specs=[
            pl.BlockSpec((_R, _D), lambda i: (i, 0)),
            pl.BlockSpec((_D, _D), lambda i: (0, 0)),
        ],
        out_specs=pl.BlockSpec((_R, _D), lambda i: (i, 0)),
        out_shape=jax.ShapeDtypeStruct((_NP, _D), jnp.float32),
    )(feat, w0)


def _prescale_body(h0_ref, din_ref, dout_ref, sin_ref, sout_ref,
                   hl_ref, hr_ref):
    din = din_ref[0] + din_ref[1]
    dout = dout_ref[0] + dout_ref[1]
    sin = jnp.broadcast_to(lax.rsqrt(jnp.maximum(din, 1.0)), (_R, _H))
    sout = jnp.broadcast_to(lax.rsqrt(jnp.maximum(dout, 1.0)), (_R, _H))
    sin_ref[...] = sin
    sout_ref[...] = sout
    h0 = h0_ref[...]
    hl_ref[...] = h0[:, :_H] * sout
    hr_ref[...] = h0[:, _H:] * sout


def _tc_prescale(h0, din, dout):
    return pl.pallas_call(
        _prescale_body,
        grid=(_NP // _R,),
        in_specs=[
            pl.BlockSpec((_R, _D), lambda i: (i, 0)),
            pl.BlockSpec((2, _R, 1), lambda i: (0, i, 0)),
            pl.BlockSpec((2, _R, 1), lambda i: (0, i, 0)),
        ],
        out_specs=[pl.BlockSpec((_R, _H), lambda i: (i, 0))] * 4,
        out_shape=[jax.ShapeDtypeStruct((_NP, _H), jnp.float32)] * 4,
    )(h0, din, dout)


def _mid_body(aggl_ref, aggr_ref, sin_ref, sout_ref, b0_ref,
              hxl_ref, hxr_ref, hxpl_ref, hxpr_ref):
    sin = sin_ref[...]
    sout = sout_ref[...]
    b0 = b0_ref[...]
    hxl = jnp.maximum(aggl_ref[...] * sin + b0[:, :_H], 0.0)
    hxr = jnp.maximum(aggr_ref[...] * sin + b0[:, _H:], 0.0)
    hxl_ref[...] = hxl
    hxr_ref[...] = hxr
    hxpl_ref[...] = hxl * sout
    hxpr_ref[...] = hxr * sout


def _tc_mid(aggl, aggr, sin, sout, b0):
    return pl.pallas_call(
        _mid_body,
        grid=(_NP // _R,),
        in_specs=[
            pl.BlockSpec((_R, _H), lambda i: (i, 0)),
            pl.BlockSpec((_R, _H), lambda i: (i, 0)),
            pl.BlockSpec((_R, _H), lambda i: (i, 0)),
            pl.BlockSpec((_R, _H), lambda i: (i, 0)),
            pl.BlockSpec((1, _D), lambda i: (0, 0)),
        ],
        out_specs=[pl.BlockSpec((_R, _H), lambda i: (i, 0))] * 4,
        out_shape=[jax.ShapeDtypeStruct((_NP, _H), jnp.float32)] * 4,
    )(aggl, aggr, sin, sout, b0)


def _final_body(hxl_ref, hxr_ref, aggl_ref, aggr_ref, sin_ref, w1_ref, b1_ref,
                out_ref):
    w1 = w1_ref[...]
    sin = sin_ref[...]
    acc = jnp.dot(hxl_ref[...], w1[:_H], preferred_element_type=jnp.float32)
    acc += jnp.dot(hxr_ref[...], w1[_H:_D], preferred_element_type=jnp.float32)
    acc += jnp.dot(aggl_ref[...] * sin, w1[_D:_D + _H],
                   preferred_element_type=jnp.float32)
    acc += jnp.dot(aggr_ref[...] * sin, w1[_D + _H:],
                   preferred_element_type=jnp.float32)
    out_ref[...] = jnp.maximum(acc + b1_ref[...], 0.0)


def _tc_final(hxl, hxr, aggl, aggr, sin, w1, b1):
    return pl.pallas_call(
        _final_body,
        grid=(_NP // _R,),
        in_specs=[
            pl.BlockSpec((_R, _H), lambda i: (i, 0)),
            pl.BlockSpec((_R, _H), lambda i: (i, 0)),
            pl.BlockSpec((_R, _H), lambda i: (i, 0)),
            pl.BlockSpec((_R, _H), lambda i: (i, 0)),
            pl.BlockSpec((_R, _H), lambda i: (i, 0)),
            pl.BlockSpec((2 * _D, _H), lambda i: (0, 0)),
            pl.BlockSpec((1, _H), lambda i: (0, 0)),
        ],
        out_specs=pl.BlockSpec((_R, _H), lambda i: (i, 0)),
        out_shape=jax.ShapeDtypeStruct((_NP, _H), jnp.float32),
    )(hxl, hxr, aggl, aggr, sin, w1, b1)


def kernel(feat, edge_index, W0, b0, W1, b1):
    pad = jnp.full((2, _EP - _E), _N, jnp.int32)
    edge_p = jnp.concatenate([edge_index, pad], axis=1)
    src = edge_p[0].reshape(_NWIN, _W)
    dst = edge_p[1].reshape(_NWIN, _W)
    feat_p = jnp.pad(feat, ((0, _NP - _N), (0, 0)))
    ones_w = jnp.ones((_W,), jnp.float32)
    zer_1d = jnp.zeros((_STRIPE,), jnp.float32)
    zer_2d = jnp.zeros((_W, _H), jnp.float32)
    b0_2d = b0.reshape(1, _D)
    b1_2d = b1.reshape(1, _H)

    h0 = _tc_matmul0(feat_p, W0)
    din_p, dout_p = _sc_degrees(src, dst, ones_w, zer_1d)
    sin, sout, hl, hr = _tc_prescale(
        h0, din_p.reshape(2, _NP, 1), dout_p.reshape(2, _NP, 1))
    aggl0, aggr0 = _sc_aggregate(hl, hr, src, dst, zer_2d)
    hxl, hxr, hxpl, hxpr = _tc_mid(aggl0, aggr0, sin, sout, b0_2d)
    aggl1, aggr1 = _sc_aggregate(hxpl, hxpr, src, dst, zer_2d)
    out = _tc_final(hxl, hxr, aggl1, aggr1, sin, W1, b1_2d)
    return out[:_N]
